# baseline (speedup 1.0000x reference)
import functools

import jax
import jax.numpy as jnp
from jax.experimental import pallas as pl
from jax.experimental.pallas import tpu as pltpu

DIM = 1024
HEADS = 16
DH = 64
E = 8
DFF = 1024
S = 2048

SBLK = 256
NSB = S // SBLK
NCHUNK = 2 * HEADS
CPAD = 128
NBMAX = 24
P = NBMAX * SBLK
VAUG = 128


def _qkv_router_body(x_ref, wqkv_ref, pool_ref, exg_ref, wg_ref, ls_ref,
                     qkv_ref, a_ref, rank_ref, i1_ref, i2_ref,
                     g12_ref, cnt_ref):
    sblk = pl.program_id(0)
    x = x_ref[...]
    x16 = x.astype(jnp.bfloat16)
    qkv = jnp.dot(x16, wqkv_ref[...], preferred_element_type=jnp.float32)
    qk = qkv[:, : 2 * HEADS * DH]
    v = qkv[:, 2 * HEADS * DH :]
    sq = (qk * qk).astype(jnp.bfloat16)
    ss = jnp.dot(sq, pool_ref[...], preferred_element_type=jnp.float32)
    rec = (1.0 / (jnp.sqrt(ss) + 1e-6)).astype(jnp.bfloat16)
    recb = jnp.dot(rec, exg_ref[...], preferred_element_type=jnp.float32)
    qkn = (qk * recb).astype(jnp.bfloat16)
    qkv_ref[...] = jnp.concatenate([qkn, v.astype(jnp.bfloat16)], axis=1)

    logits = jnp.dot(x, wg_ref[...], preferred_element_type=jnp.float32)
    eids = jax.lax.broadcasted_iota(jnp.int32, (SBLK, E), 1)
    m1 = jnp.max(logits, axis=-1, keepdims=True)
    a1 = jnp.argmax(logits, axis=-1)[:, None]
    masked = jnp.where(eids == a1, -jnp.inf, logits)
    m2 = jnp.max(masked, axis=-1, keepdims=True)
    a2 = jnp.argmax(masked, axis=-1)[:, None]
    z = jnp.exp(m2 - m1)
    g1 = 1.0 / (1.0 + z)
    g2 = z / (1.0 + z)
    i1 = (eids == a1).astype(jnp.float32)
    i2 = (eids == a2).astype(jnp.float32)
    a = i1 + i2
    i1_ref[...] = i1
    i2_ref[...] = i2
    a_ref[...] = a
    g12_ref[...] = jnp.where(eids == 0, g1, jnp.where(eids == 1, g2, 0.0))

    @pl.when(sblk == 0)
    def _zero():
        cnt_ref[...] = jnp.zeros_like(cnt_ref)

    cnt = cnt_ref[0:1, 0:E]
    rank_ref[...] = jnp.dot(ls_ref[...], a.astype(jnp.bfloat16),
                            preferred_element_type=jnp.float32) + cnt
    cnt_ref[0:1, 0:E] = cnt + jnp.sum(a, axis=0, keepdims=True)


def _qkv_router(x, wqkv, pool, exg, wg, ls):
    return pl.pallas_call(
        _qkv_router_body,
        grid=(NSB,),
        in_specs=[
            pl.BlockSpec((SBLK, DIM), lambda s: (s, 0)),
            pl.BlockSpec((DIM, 3 * HEADS * DH), lambda s: (0, 0)),
            pl.BlockSpec((2 * HEADS * DH, CPAD), lambda s: (0, 0)),
            pl.BlockSpec((CPAD, 2 * HEADS * DH), lambda s: (0, 0)),
            pl.BlockSpec((DIM, E), lambda s: (0, 0)),
            pl.BlockSpec((SBLK, SBLK), lambda s: (0, 0)),
        ],
        out_specs=[
            pl.BlockSpec((SBLK, 3 * HEADS * DH), lambda s: (s, 0)),
            pl.BlockSpec((SBLK, E), lambda s: (s, 0)),
            pl.BlockSpec((SBLK, E), lambda s: (s, 0)),
            pl.BlockSpec((SBLK, E), lambda s: (s, 0)),
            pl.BlockSpec((SBLK, E), lambda s: (s, 0)),
            pl.BlockSpec((SBLK, E), lambda s: (s, 0)),
        ],
        out_shape=[
            jax.ShapeDtypeStruct((S, 3 * HEADS * DH), jnp.bfloat16),
            jax.ShapeDtypeStruct((S, E), jnp.float32),
            jax.ShapeDtypeStruct((S, E), jnp.float32),
            jax.ShapeDtypeStruct((S, E), jnp.float32),
            jax.ShapeDtypeStruct((S, E), jnp.float32),
            jax.ShapeDtypeStruct((S, E), jnp.float32),
        ],
        scratch_shapes=[pltpu.VMEM((8, 128), jnp.float32)],
    )(x, wqkv, pool, exg, wg, ls)


def _dispatch_body(a_ref, rank_ref, i1_ref, i2_ref, tri_ref,
                   pos1_ref, pos2_ref, cnts_ref):
    a = a_ref[...]
    counts = jnp.sum(a, axis=0, keepdims=True)
    padded = jnp.floor((counts + (SBLK - 1.0)) / SBLK) * SBLK
    start = jnp.dot(padded.astype(jnp.bfloat16), tri_ref[...],
                    preferred_element_type=jnp.float32)
    pos = rank_ref[...] + start
    pos1_ref[...] = jnp.sum(pos * i1_ref[...], axis=1,
                            keepdims=True).astype(jnp.int32)
    pos2_ref[...] = jnp.sum(pos * i2_ref[...], axis=1,
                            keepdims=True).astype(jnp.int32)
    cnts_ref[...] = counts


def _dispatch(a, rank, i1, i2, tri):
    return pl.pallas_call(
        _dispatch_body,
        in_specs=[pl.BlockSpec((S, E), lambda: (0, 0))] * 4
        + [pl.BlockSpec((E, E), lambda: (0, 0))],
        out_specs=[
            pl.BlockSpec((S, 1), lambda: (0, 0)),
            pl.BlockSpec((S, 1), lambda: (0, 0)),
            pl.BlockSpec((1, E), lambda: (0, 0)),
        ],
        out_shape=[
            jax.ShapeDtypeStruct((S, 1), jnp.int32),
            jax.ShapeDtypeStruct((S, 1), jnp.int32),
            jax.ShapeDtypeStruct((1, E), jnp.float32),
        ],
    )(a, rank, i1, i2, tri)


def _poly_exp(s):
    t = s * (1.0 / 24.0) + (1.0 / 6.0)
    t = t * s + 0.5
    t = t * s + 1.0
    return t * s + 1.0


def _attn_body(q_ref, k_ref, v_ref, o_ref):
    for qb in range(NSB):
        q = q_ref[0, pl.ds(qb * SBLK, SBLK), :]
        acc = jnp.zeros((SBLK, VAUG), jnp.float32)
        for j in range(qb + 1):
            k = k_ref[0, pl.ds(j * SBLK, SBLK), :]
            s = jax.lax.dot_general(q, k, (((1,), (1,)), ((), ())),
                                    preferred_element_type=jnp.float32)
            p = _poly_exp(s)
            if j == qb:
                row = jax.lax.broadcasted_iota(jnp.int32, (SBLK, SBLK), 0)
                col = jax.lax.broadcasted_iota(jnp.int32, (SBLK, SBLK), 1)
                p = jnp.where(col <= row, p, 0.0)
            vv = v_ref[0, pl.ds(j * SBLK, SBLK), :]
            acc = acc + jnp.dot(p.astype(jnp.bfloat16), vv,
                                preferred_element_type=jnp.float32)
        o_ref[0, pl.ds(qb * SBLK, SBLK), :] = (
            acc[:, :DH] / acc[:, DH : DH + 1]).astype(jnp.bfloat16)


def _flash_attn(q, k, vaug):
    return pl.pallas_call(
        _attn_body,
        grid=(HEADS,),
        in_specs=[
            pl.BlockSpec((1, S, DH), lambda h: (h, 0, 0)),
            pl.BlockSpec((1, S, DH), lambda h: (h, 0, 0)),
            pl.BlockSpec((1, S, VAUG), lambda h: (h, 0, 0)),
        ],
        out_specs=pl.BlockSpec((1, S, DH), lambda h: (h, 0, 0)),
        out_shape=jax.ShapeDtypeStruct((HEADS, S, DH), jnp.bfloat16),
    )(q, k, vaug)


def _expert_body(be_ref, x_ref, w1_ref, w2_ref, y_ref):
    h = jnp.dot(x_ref[...], w1_ref[0], preferred_element_type=jnp.float32)
    h = h * jax.lax.logistic(h)
    y_ref[...] = jnp.dot(h, w2_ref[0], preferred_element_type=jnp.float32)


def _experts(block_expert, x_disp, w1, w2):
    grid_spec = pltpu.PrefetchScalarGridSpec(
        num_scalar_prefetch=1,
        grid=(NBMAX,),
        in_specs=[
            pl.BlockSpec((SBLK, DIM), lambda i, be: (i, 0)),
            pl.BlockSpec((1, DIM, DFF), lambda i, be: (be[i], 0, 0)),
            pl.BlockSpec((1, DFF, DIM), lambda i, be: (be[i], 0, 0)),
        ],
        out_specs=pl.BlockSpec((SBLK, DIM), lambda i, be: (i, 0)),
    )
    return pl.pallas_call(
        _expert_body,
        grid_spec=grid_spec,
        out_shape=jax.ShapeDtypeStruct((P, DIM), jnp.float32),
    )(block_expert, x_disp, w1, w2)


def _proj_res_body(o_ref, wo_ref, x_ref, y1_ref, y2_ref, g12_ref, out_ref):
    g12 = g12_ref[...]
    y = g12[:, 0:1] * y1_ref[...] + g12[:, 1:2] * y2_ref[...]
    out_ref[...] = x_ref[...] + y + jnp.dot(
        o_ref[...], wo_ref[...], preferred_element_type=jnp.float32)


def _out_proj_combine(o, wo, x, y1, y2, g12):
    return pl.pallas_call(
        _proj_res_body,
        grid=(NSB,),
        in_specs=[
            pl.BlockSpec((SBLK, HEADS * DH), lambda s: (s, 0)),
            pl.BlockSpec((HEADS * DH, DIM), lambda s: (0, 0)),
            pl.BlockSpec((SBLK, DIM), lambda s: (s, 0)),
            pl.BlockSpec((SBLK, DIM), lambda s: (s, 0)),
            pl.BlockSpec((SBLK, DIM), lambda s: (s, 0)),
            pl.BlockSpec((SBLK, E), lambda s: (s, 0)),
        ],
        out_specs=pl.BlockSpec((SBLK, DIM), lambda s: (s, 0)),
        out_shape=jax.ShapeDtypeStruct((S, DIM), jnp.float32),
    )(o, wo, x, y1, y2, g12)


def _norm_consts(qg, kg):
    j = jnp.arange(2 * HEADS * DH)
    c = jnp.arange(CPAD)
    onehot = (j[:, None] // DH) == c[None, :]
    pool = onehot.astype(jnp.bfloat16)
    gains = jnp.concatenate([qg.reshape(-1) * (DH ** -0.5), kg.reshape(-1)])
    exg = (onehot.T * gains[None, :]).astype(jnp.bfloat16)
    return pool, exg


def kernel(x, Wq, Wk, Wv, Wo, qg, kg, Wg, w1, w2):
    b, s, d = x.shape
    xt = x.reshape(s, d)
    ii = jnp.arange(SBLK)
    ls = (ii[:, None] > ii[None, :]).astype(jnp.bfloat16)
    ee = jnp.arange(E)
    tri = (ee[:, None] < ee[None, :]).astype(jnp.bfloat16)
    tok = jnp.arange(S, dtype=jnp.int32)
    for l in range(Wq.shape[0]):
        wqkv = jnp.concatenate([Wq[l], Wk[l], Wv[l]], axis=1).astype(jnp.bfloat16)
        pool, exg = _norm_consts(qg[l], kg[l])
        qkv, a, rank, i1, i2, g12 = _qkv_router(
            xt, wqkv, pool, exg, Wg[l], ls)
        pos1, pos2, counts = _dispatch(a, rank, i1, i2, tri)
        pos1, pos2 = pos1.reshape(S), pos2.reshape(S)
        nb = jnp.ceil(counts.reshape(E) / SBLK).astype(jnp.int32)
        block_expert = jnp.repeat(jnp.arange(E, dtype=jnp.int32), nb,
                                  total_repeat_length=NBMAX)
        tid = jnp.zeros((P,), jnp.int32).at[pos1].set(tok).at[pos2].set(tok)
        x_disp = jnp.take(xt, tid, axis=0)
        y_disp = _experts(block_expert, x_disp, w1[l], w2[l])
        y1 = jnp.take(y_disp, pos1, axis=0)
        y2 = jnp.take(y_disp, pos2, axis=0)

        qkv3 = qkv.reshape(S, 3 * HEADS, DH)
        q = qkv3[:, :HEADS, :].transpose(1, 0, 2)
        k = qkv3[:, HEADS : 2 * HEADS, :].transpose(1, 0, 2)
        v = qkv3[:, 2 * HEADS :, :].transpose(1, 0, 2)
        vaug = jnp.concatenate(
            [v, jnp.ones((HEADS, S, 1), jnp.bfloat16),
             jnp.zeros((HEADS, S, VAUG - DH - 1), jnp.bfloat16)], axis=-1)
        o = _flash_attn(q, k, vaug)
        o2 = o.transpose(1, 0, 2).reshape(S, HEADS * DH)
        xt = _out_proj_combine(o2, Wo[l].astype(jnp.bfloat16), xt, y1, y2, g12)
    return xt.reshape(b, s, d)
